# baseline (speedup 1.0000x reference)
import functools

import jax
import jax.numpy as jnp
from jax import lax
from jax.experimental import pallas as pl
from jax.experimental.pallas import tpu as pltpu
from jax.experimental.pallas import tpu_sc as plsc

_L = 16
_W = 128
_CW = 128


def _sc_transpose(E_user, E_city, tail_u, tail_c):
    nu, du = E_user.shape
    nc, dc = E_city.shape

    ub_rows, cb_rows = 512, 1024
    nb_u = nu // ub_rows
    nb_c = nc // cb_rows
    info = plsc.get_sparse_core_info()
    nw = info.num_cores * info.num_subcores
    ou_rows = nu * du // _W
    oc_rows = nb_c * _W + tail_c.shape[0]

    mesh = plsc.VectorSubcoreMesh(core_axis_name="c", subcore_axis_name="s")

    @functools.partial(
        pl.kernel,
        mesh=mesh,
        compiler_params=pltpu.CompilerParams(needs_layout_passes=False),
        out_type=(
            jax.ShapeDtypeStruct((ou_rows, _W), jnp.float32),
            jax.ShapeDtypeStruct((oc_rows, _W), jnp.float32),
        ),
        scratch_types=[
            pltpu.VMEM((du, ub_rows), jnp.float32),
            pltpu.VMEM((du, ub_rows), jnp.float32),
            pltpu.VMEM((dc, cb_rows), jnp.float32),
            pltpu.VMEM((_W, _W), jnp.float32),
            pltpu.VMEM((_W, _W), jnp.float32),
            pltpu.SemaphoreType.DMA,
            pltpu.SemaphoreType.DMA,
            pltpu.SemaphoreType.DMA,
            pltpu.SemaphoreType.DMA,
        ],
    )
    def body(eu_t, ec_t, tu_h, tc_h, ou_h, oc_h,
             ub0, ub1, cb, ob0, ob1, si0, si1, so0, so1):
        wid = lax.axis_index("s") * info.num_cores + lax.axis_index("c")
        ubs, obs = (ub0, ub1), (ob0, ob1)
        sin, sout = (si0, si1), (so0, so1)
        iota = lax.iota(jnp.int32, _L)

        def transpose_block(src, dst, d, n_groups):
            lgd = d.bit_length() - 1

            def grp(g, c):
                if d == 32:
                    tau = iota * 0 + lax.shift_right_logical(g, 1)
                    jv = iota + lax.shift_left(g & 1, 4)
                else:
                    tau = iota * 0 + g
                    jv = iota
                data = plsc.load_gather(src, [jv, tau])
                row = lax.shift_right_logical(g, 3)
                col = lax.shift_left(g & 7, 4)
                dst[row, pl.ds(col, _L)] = data
                return c

            lax.fori_loop(0, n_groups, grp, 0, unroll=8)

        n_steps = nb_u // nw
        in_cp, out_cp = {}, {}

        def u_start_in(s, t):
            cp = pltpu.make_async_copy(
                eu_t.at[:, pl.ds(pl.multiple_of(t * ub_rows, 128), ub_rows)],
                ubs[s % 2], sin[s % 2])
            in_cp[s] = cp
            cp.start()

        def u_start_out(s, t):
            cp = pltpu.make_async_copy(
                obs[s % 2], ou_h.at[pl.ds(pl.multiple_of(t * _W, 8), _W)],
                sout[s % 2])
            out_cp[s] = cp
            cp.start()

        u_start_in(0, wid)
        u_start_in(1, nw + wid)
        for s in range(n_steps):
            in_cp[s].wait()
            if s >= 2:
                out_cp[s - 2].wait()
            transpose_block(ubs[s % 2], obs[s % 2], du, ub_rows * du // _L)
            if s + 2 < n_steps:
                u_start_in(s + 2, (s + 2) * nw + wid)
            u_start_out(s, s * nw + wid)
        out_cp[n_steps - 2].wait()
        out_cp[n_steps - 1].wait()

        @pl.when(wid == 0)
        def _():
            pltpu.sync_copy(eu_t.at[:, pl.ds(nb_u // nw * nw * ub_rows,
                                             ub_rows)], ub0)
            transpose_block(ub0, ob0, du, ub_rows * du // _L)
            pltpu.sync_copy(ob0, ou_h.at[pl.ds(nb_u // nw * nw * _W, _W)])

        for s in range(nb_c // nw):
            t = s * nw + wid
            pltpu.sync_copy(
                ec_t.at[:, pl.ds(pl.multiple_of(t * cb_rows, 128),
                                 cb_rows)], cb)
            transpose_block(cb, ob0, dc, cb_rows * dc // _L)
            pltpu.sync_copy(
                ob0, oc_h.at[pl.ds(pl.multiple_of(t * _W, 8), _W)])

        @pl.when(wid == 1)
        def _():
            pltpu.sync_copy(ec_t.at[:, pl.ds(nb_c // nw * nw * cb_rows,
                                             cb_rows)], cb)
            transpose_block(cb, ob1, dc, cb_rows * dc // _L)
            pltpu.sync_copy(ob1, oc_h.at[pl.ds(nb_c // nw * nw * _W, _W)])

        tu_n = tail_u.shape[0]
        tc_n = tail_c.shape[0]

        @pl.when(wid == 4)
        def _():
            pltpu.sync_copy(tu_h, ob1.at[pl.ds(0, tu_n)])
            pltpu.sync_copy(ob1.at[pl.ds(0, tu_n)],
                            ou_h.at[pl.ds(nb_u // nw * nw * _W + _W, tu_n)])

        @pl.when(wid == 2)
        def _():
            pltpu.sync_copy(tc_h, ob1.at[pl.ds(0, tc_n)])
            pltpu.sync_copy(ob1.at[pl.ds(0, tc_n)],
                            oc_h.at[pl.ds(nb_c // nw * nw * _W + _W, tc_n)])

    return body(E_user.T, E_city.T, tail_u, tail_c)


def _sc_gather(user_id, city_id, device_id, wu, wc, wd, dims):
    B = user_id.shape[0]
    info = plsc.get_sparse_core_info()
    nw = info.num_cores * info.num_subcores
    per_w = B // nw
    n_ch = per_w // _CW

    mesh = plsc.VectorSubcoreMesh(core_axis_name="c", subcore_axis_name="s")

    @functools.partial(
        pl.kernel,
        mesh=mesh,
        compiler_params=pltpu.CompilerParams(needs_layout_passes=False),
        out_type=tuple(
            jax.ShapeDtypeStruct((nw, per_w * d // _W, _W), jnp.float32)
            for d in dims),
        scratch_types=[
            pltpu.VMEM((per_w,), jnp.int32),
            pltpu.VMEM((per_w,), jnp.int32),
            pltpu.VMEM((per_w,), jnp.int32),
            pltpu.VMEM((per_w,), jnp.int32),
            pltpu.VMEM((per_w,), jnp.int32),
            pltpu.VMEM((per_w,), jnp.int32),
            pltpu.VMEM((_CW, _W), jnp.float32),
            pltpu.VMEM((_CW, _W), jnp.float32),
            pltpu.VMEM((per_w * dims[0] // _W, _W), jnp.float32),
            pltpu.VMEM((per_w * dims[1] // _W, _W), jnp.float32),
            pltpu.VMEM((per_w * dims[2] // _W, _W), jnp.float32),
            pltpu.SemaphoreType.DMA,
        ],
    )
    def body(uid_h, cid_h, did_h, eu_h, ec_h, ed_h, ou_h, oc_h, od_h,
             i0, i1, i2, p0, p1, p2, w0, w1, b0, b1, b2, sem):
        wid = lax.axis_index("s") * info.num_cores + lax.axis_index("c")
        base = wid * per_w
        idx_refs = (i0, i1, i2)
        prow_refs = (p0, p1, p2)
        wide_refs = (w0, w1)
        out_vrefs = (b0, b1, b2)
        out_hrefs = (ou_h, oc_h, od_h)
        tab_hrefs = (eu_h, ec_h, ed_h)

        for i_ref, src in zip(idx_refs, (uid_h, cid_h, did_h)):
            pltpu.sync_copy(src.at[pl.ds(base, per_w)], i_ref)

        for t in range(3):
            sh = 2 if dims[t] == 32 else 3
            i_ref, p_ref = idx_refs[t], prow_refs[t]

            def prow_step(g, c, i_ref=i_ref, p_ref=p_ref, sh=sh):
                sl = pl.ds(g * _L, _L)
                p_ref[sl] = lax.shift_right_logical(i_ref[sl], sh)
                return c

            lax.fori_loop(0, per_w // _L, prow_step, 0)

        chunks = [(t, j) for t in range(3) for j in range(n_ch)]

        def start(c):
            t, j = chunks[c]
            sl = pl.ds(j * _CW, _CW)
            return pltpu.async_copy(
                tab_hrefs[t].at[prow_refs[t].at[sl]],
                wide_refs[c % 2], sem)

        def extract(c):
            t, j = chunks[c]
            d = dims[t]
            lg = 5 if d == 32 else 4
            msk_r = (1 << (7 - lg)) - 1
            wide = wide_refs[c % 2]
            i_ref = idx_refs[t]
            buf = out_vrefs[t]
            n_g = _CW * d // _L
            k_base = j * _CW * d
            iota = lax.iota(jnp.int32, _L)

            def grp(g, c2):
                k0 = g * _L
                kv = k0 + iota
                r = lax.shift_right_logical(kv, lg)
                iv = plsc.load_gather(i_ref, [j * _CW + r])
                off = lax.shift_left(iv & msk_r, lg)
                col = off + (kv & (d - 1))
                data = plsc.load_gather(wide, [r, col])
                kg = k_base + k0
                buf[lax.shift_right_logical(kg, 7),
                    pl.ds(kg & (_W - 1), _L)] = data
                return c2

            lax.fori_loop(0, n_g, grp, 0, unroll=8)

        cps = {0: start(0)}
        for c in range(len(chunks)):
            if c + 1 < len(chunks):
                cps[c + 1] = start(c + 1)
            cps[c].wait()
            extract(c)

        for t in range(3):
            pltpu.sync_copy(out_vrefs[t], out_hrefs[t].at[wid])

    ou, oc, od = body(user_id, city_id, device_id, wu, wc, wd)
    return (ou.reshape(B, dims[0]), oc.reshape(B, dims[1]),
            od.reshape(B, dims[2]))


def _mlp_body(eu_r, ec_r, ed_r, us_r, wd_r, bd_r, w1_r, b1_r, w2_r, b2_r,
              w3_r, b3_r, out_r):
    f32 = jnp.float32
    dense = jnp.dot(us_r[...], wd_r[...], preferred_element_type=f32) + bd_r[...]
    w1 = w1_r[...]
    h = (jnp.dot(eu_r[...], w1[0:32, :], preferred_element_type=f32)
         + jnp.dot(ec_r[...], w1[32:48, :], preferred_element_type=f32)
         + jnp.dot(ed_r[...], w1[48:64, :], preferred_element_type=f32)
         + jnp.dot(dense, w1[64:96, :], preferred_element_type=f32)
         + b1_r[...])
    h = jnp.maximum(h, 0.0)
    h = jnp.maximum(jnp.dot(h, w2_r[...], preferred_element_type=f32)
                    + b2_r[...], 0.0)
    out_r[...] = jnp.dot(h, w3_r[...], preferred_element_type=f32) + b3_r[...]


def _mlp(eu, ec, ed, user_stats, W_dense, b_dense, W1, b1, W2, b2, W3, b3):
    B = eu.shape[0]
    blk = 2048
    grid = (B // blk,)
    full = lambda shape: pl.BlockSpec(shape, lambda i: (0, 0))
    batched = lambda d: pl.BlockSpec((blk, d), lambda i: (i, 0))
    return pl.pallas_call(
        _mlp_body,
        grid=grid,
        in_specs=[
            batched(eu.shape[1]),
            batched(ec.shape[1]),
            batched(ed.shape[1]),
            batched(user_stats.shape[1]),
            full(W_dense.shape),
            full((1, b_dense.shape[0])),
            full(W1.shape),
            full((1, b1.shape[0])),
            full(W2.shape),
            full((1, b2.shape[0])),
            full(W3.shape),
            full((1, b3.shape[0])),
        ],
        out_specs=batched(W3.shape[1]),
        out_shape=jax.ShapeDtypeStruct((B, W3.shape[1]), jnp.float32),
    )(eu, ec, ed, user_stats, W_dense, b_dense.reshape(1, -1), W1,
      b1.reshape(1, -1), W2, b2.reshape(1, -1), W3, b3.reshape(1, -1))


def kernel(user_id, city_id, device_id, user_stats, E_user, E_city, E_dev,
           W_dense, b_dense, W1, b1, W2, b2, W3, b3):
    dims = (E_user.shape[1], E_city.shape[1], E_dev.shape[1])
    tail_u = E_user[1953 * 512:].reshape(16, _W)
    tail_c = jnp.pad(E_city[97 * 1024:].reshape(84, _W), ((0, 4), (0, 0)))
    wd = jnp.pad(E_dev.reshape(125, _W), ((0, 3), (0, 0)))
    wu, wc = _sc_transpose(E_user, E_city, tail_u, tail_c)
    eu, ec, ed = _sc_gather(user_id, city_id, device_id, wu, wc, wd, dims)
    return _mlp(eu, ec, ed, user_stats, W_dense, b_dense, W1, b1, W2, b2,
                W3, b3)


# speedup vs baseline: 1.5891x; 1.5891x over previous
import functools

import jax
import jax.numpy as jnp
from jax import lax
from jax.experimental import pallas as pl
from jax.experimental.pallas import tpu as pltpu
from jax.experimental.pallas import tpu_sc as plsc

_CW = 128


def _sc_gather(user_id, city_id, device_id, E_user, E_city, E_dev):
    B = user_id.shape[0]
    info = plsc.get_sparse_core_info()
    nw = info.num_cores * info.num_subcores
    per_w = B // nw
    ch = per_w // _CW
    du = E_user.shape[1]
    dc = E_city.shape[1]
    dd = E_dev.shape[1]

    uid = user_id.reshape(nw, ch, _CW)
    cid = city_id.reshape(nw, ch, _CW)
    did = device_id.reshape(nw, ch, _CW)

    mesh = plsc.VectorSubcoreMesh(core_axis_name="c", subcore_axis_name="s")

    @functools.partial(
        pl.kernel,
        mesh=mesh,
        compiler_params=pltpu.CompilerParams(use_tc_tiling_on_sc=False),
        out_type=(
            jax.ShapeDtypeStruct((nw, per_w, du), jnp.float32),
            jax.ShapeDtypeStruct((nw, per_w, dc), jnp.float32),
            jax.ShapeDtypeStruct((nw, per_w, dd), jnp.float32),
        ),
        scratch_types=[
            pltpu.VMEM((ch, _CW), jnp.int32),
            pltpu.VMEM((ch, _CW), jnp.int32),
            pltpu.VMEM((ch, _CW), jnp.int32),
            pltpu.VMEM((per_w, du), jnp.float32),
            pltpu.VMEM((per_w, dc), jnp.float32),
            pltpu.VMEM((per_w, dd), jnp.float32),
            pltpu.SemaphoreType.DMA,
        ],
    )
    def body(uid_h, cid_h, did_h, eu_h, ec_h, ed_h, ou_h, oc_h, od_h,
             iu, ic, idv, ru, rc, rd, sem):
        wid = lax.axis_index("s") * info.num_cores + lax.axis_index("c")
        pltpu.sync_copy(uid_h.at[wid], iu)
        pltpu.sync_copy(cid_h.at[wid], ic)
        pltpu.sync_copy(did_h.at[wid], idv)
        copies = []
        for j in range(ch):
            sl = pl.ds(j * _CW, _CW)
            copies.append(pltpu.async_copy(eu_h.at[iu.at[j]], ru.at[sl], sem))
            copies.append(pltpu.async_copy(ec_h.at[ic.at[j]], rc.at[sl], sem))
            copies.append(pltpu.async_copy(ed_h.at[idv.at[j]], rd.at[sl], sem))
        for c in copies:
            c.wait()
        pltpu.sync_copy(ru, ou_h.at[wid])
        pltpu.sync_copy(rc, oc_h.at[wid])
        pltpu.sync_copy(rd, od_h.at[wid])

    ou, oc, od = body(uid, cid, did, E_user, E_city, E_dev)
    return ou.reshape(B, du), oc.reshape(B, dc), od.reshape(B, dd)


def _mlp_body(eu_r, ec_r, ed_r, us_r, wd_r, bd_r, w1_r, b1_r, w2_r, b2_r,
              w3_r, b3_r, out_r):
    f32 = jnp.float32
    dense = jnp.dot(us_r[...], wd_r[...], preferred_element_type=f32) + bd_r[...]
    w1 = w1_r[...]
    h = (jnp.dot(eu_r[...], w1[0:32, :], preferred_element_type=f32)
         + jnp.dot(ec_r[...], w1[32:48, :], preferred_element_type=f32)
         + jnp.dot(ed_r[...], w1[48:64, :], preferred_element_type=f32)
         + jnp.dot(dense, w1[64:96, :], preferred_element_type=f32)
         + b1_r[...])
    h = jnp.maximum(h, 0.0)
    h = jnp.maximum(jnp.dot(h, w2_r[...], preferred_element_type=f32)
                    + b2_r[...], 0.0)
    out_r[...] = jnp.dot(h, w3_r[...], preferred_element_type=f32) + b3_r[...]


def _mlp(eu, ec, ed, user_stats, W_dense, b_dense, W1, b1, W2, b2, W3, b3):
    B = eu.shape[0]
    blk = 2048
    grid = (B // blk,)
    full = lambda shape: pl.BlockSpec(shape, lambda i: (0, 0))
    batched = lambda d: pl.BlockSpec((blk, d), lambda i: (i, 0))
    return pl.pallas_call(
        _mlp_body,
        grid=grid,
        in_specs=[
            batched(eu.shape[1]),
            batched(ec.shape[1]),
            batched(ed.shape[1]),
            batched(user_stats.shape[1]),
            full(W_dense.shape),
            full((1, b_dense.shape[0])),
            full(W1.shape),
            full((1, b1.shape[0])),
            full(W2.shape),
            full((1, b2.shape[0])),
            full(W3.shape),
            full((1, b3.shape[0])),
        ],
        out_specs=batched(W3.shape[1]),
        out_shape=jax.ShapeDtypeStruct((B, W3.shape[1]), jnp.float32),
    )(eu, ec, ed, user_stats, W_dense, b_dense.reshape(1, -1), W1,
      b1.reshape(1, -1), W2, b2.reshape(1, -1), W3, b3.reshape(1, -1))


def kernel(user_id, city_id, device_id, user_stats, E_user, E_city, E_dev,
           W_dense, b_dense, W1, b1, W2, b2, W3, b3):
    eu, ec, ed = _sc_gather(user_id, city_id, device_id, E_user, E_city, E_dev)
    return _mlp(eu, ec, ed, user_stats, W_dense, b_dense, W1, b1, W2, b2,
                W3, b3)
